# baseline (speedup 1.0000x reference)
import functools

import jax
import jax.numpy as jnp
from jax import lax
from jax.experimental import pallas as pl
from jax.experimental.pallas import tpu as pltpu
from jax.experimental.pallas import tpu_sc as plsc

N = 10000
E = 320000
D = 128
DH = 64
HID = 64
G = 64

NC = 2
NS = 16
NW = NC * NS
EN = E + N
C = 10320
EN_PAD = NW * C
B = 80
NB = C // B
NP = 10240
RPT = NP // NS
F32 = jnp.float32
I32 = jnp.int32


def _leaky(v, s):
    return jnp.where(v > 0, v, s * v)


def _tc_prep(x, ei, W, a_s, a_d):
    def body(x_ref, ei_ref, w_ref, s_ref, d_ref, lo_ref, hi_ref, hs_ref,
             hd_ref, sf_ref, df_ref):
        h = jnp.dot(x_ref[...], w_ref[...], preferred_element_type=F32)
        lo_ref[...] = h[:, :DH]
        hi_ref[...] = h[:, DH:]
        hs_ref[...] = jnp.dot(h, s_ref[...], preferred_element_type=F32)[:, 0]
        hd_ref[...] = jnp.dot(h, d_ref[...], preferred_element_type=F32)[:, 0]
        loop = lax.broadcasted_iota(I32, (N,), 0)
        padz = jnp.zeros((EN_PAD - EN,), I32)
        sf_ref[...] = jnp.concatenate([ei_ref[0], loop, padz])
        df_ref[...] = jnp.concatenate([ei_ref[1], loop, padz])
    return pl.pallas_call(
        body,
        out_shape=(
            jax.ShapeDtypeStruct((N, DH), F32),
            jax.ShapeDtypeStruct((N, DH), F32),
            jax.ShapeDtypeStruct((N,), F32),
            jax.ShapeDtypeStruct((N,), F32),
            jax.ShapeDtypeStruct((EN_PAD,), I32),
            jax.ShapeDtypeStruct((EN_PAD,), I32),
        ),
    )(x, ei, W, a_s.reshape(D, 1), a_d.reshape(D, 1))


def _combine(u_ref, den_ref, b_ref):
    dsum = jnp.sum(den_ref[...], axis=0)[:N] + 1e-16
    usum = u_ref[0, :N] + u_ref[1, :N]
    return usum / dsum[:, None] + b_ref[...]


def _tc_mid(U, den, b, W, a_s, a_d):
    def body(u_ref, den_ref, b_ref, w_ref, s_ref, d_ref, lo_ref, hi_ref,
             hs_ref, hd_ref):
        h1 = _leaky(_combine(u_ref, den_ref, b_ref), 0.01)
        h = jnp.dot(h1, w_ref[...], preferred_element_type=F32)
        lo_ref[...] = h[:, :DH]
        hi_ref[...] = h[:, DH:]
        hs_ref[...] = jnp.dot(h, s_ref[...], preferred_element_type=F32)[:, 0]
        hd_ref[...] = jnp.dot(h, d_ref[...], preferred_element_type=F32)[:, 0]
    return pl.pallas_call(
        body,
        out_shape=(
            jax.ShapeDtypeStruct((N, DH), F32),
            jax.ShapeDtypeStruct((N, DH), F32),
            jax.ShapeDtypeStruct((N,), F32),
            jax.ShapeDtypeStruct((N,), F32),
        ),
    )(U, den, b.reshape(1, D), W, a_s.reshape(D, 1), a_d.reshape(D, 1))


def _tc_pool(U, den, b, batch, Wg, bg, Wl1, bl1, Wl2, bl2):
    def body(u_ref, den_ref, b_ref, bat_ref, wg_ref, bg_ref, w1_ref, b1_ref,
             w2_ref, b2_ref, o_ref):
        h2 = _leaky(_combine(u_ref, den_ref, b_ref), 0.01)
        gate = jnp.dot(h2, wg_ref[...], preferred_element_type=F32) + bg_ref[...]
        eg = jnp.exp(gate)
        gid = lax.broadcasted_iota(I32, (G, N), 0)
        M = jnp.where(gid == bat_ref[...].reshape(1, N), 1.0, 0.0)
        pden = jnp.dot(M, eg, preferred_element_type=F32,
                       precision=lax.Precision.HIGHEST) + 1e-16
        pnum = jnp.dot(M, eg * h2, preferred_element_type=F32,
                       precision=lax.Precision.HIGHEST)
        p = pnum / pden
        q = _leaky(jnp.dot(p, w1_ref[...], preferred_element_type=F32)
                   + b1_ref[...], 0.01)
        r = _leaky(jnp.dot(q, w2_ref[...], preferred_element_type=F32)
                   + b2_ref[...], 0.01)
        o_ref[...] = r
    return pl.pallas_call(
        body,
        out_shape=jax.ShapeDtypeStruct((G, 1), F32),
    )(U, den, b.reshape(1, D), batch.reshape(1, N), Wg, bg.reshape(1, D),
      Wl1, bl1.reshape(1, HID), Wl2, bl2.reshape(1, 1))


_MESH = plsc.VectorSubcoreMesh(
    core_axis_name="c", subcore_axis_name="s", num_cores=NC, num_subcores=NS)


@functools.partial(
    pl.kernel,
    out_type=(
        jax.ShapeDtypeStruct((NC, NP, 2, DH), F32),
        jax.ShapeDtypeStruct((NW, NP), F32),
    ),
    mesh=_MESH,
    compiler_params=pltpu.CompilerParams(
        needs_layout_passes=False, use_tc_tiling_on_sc=False),
    scratch_types=(
        pltpu.VMEM((N,), F32),
        pltpu.VMEM((N,), F32),
        pltpu.VMEM((NP,), F32),
        pltpu.VMEM((NB, B), I32),
        pltpu.VMEM((NB, B), I32),
        pltpu.VMEM((C,), F32),
        pltpu.VMEM((B, DH), F32),
        pltpu.VMEM((B, DH), F32),
        pltpu.VMEM((B, DH), F32),
        pltpu.VMEM_SHARED((NP, DH), F32),
        pltpu.SemaphoreType.DMA,
        pltpu.SemaphoreType.DMA,
        pltpu.SemaphoreType.DMA,
        pltpu.SemaphoreType.DMA,
        pltpu.SemaphoreType.DMA,
        pltpu.SemaphoreType.DMA,
    ),
)
def _sc_layer(hlo_hbm, hhi_hbm, asrc_hbm, adst_hbm, src2_hbm, dst2_hbm,
              zU_hbm, zden_hbm, U_out, den_out,
              asrcv, adstv, denv, src2v, dst2v, exv, rows0, rows1, rows2,
              U_sh, gs0, gs1, gs2, ss0, ss1, ss2):
    bufs = (rows0, rows1, rows2)
    gsems = (gs0, gs1, gs2)
    ssems = (ss0, ss1, ss2)
    cid = lax.axis_index("c")
    sid = lax.axis_index("s")
    wid = sid * NC + cid
    pltpu.sync_copy(asrc_hbm, asrcv)
    pltpu.sync_copy(adst_hbm, adstv)
    pltpu.sync_copy(zden_hbm, denv)
    pltpu.sync_copy(src2_hbm.at[wid], src2v)
    pltpu.sync_copy(dst2_hbm.at[wid], dst2v)
    pltpu.sync_copy(zU_hbm, U_sh.at[pl.ds(sid * RPT, RPT)])
    plsc.subcore_barrier()

    base = wid * C

    def body_a(j, carry):
        for k in range(B // 16):
            sv = src2v[j, pl.ds(k * 16, 16)]
            dv = dst2v[j, pl.ds(k * 16, 16)]
            al = plsc.load_gather(asrcv, [sv]) + plsc.load_gather(adstv, [dv])
            ex = jnp.exp(_leaky(al, 0.2))
            off = pl.multiple_of(j * B + k * 16, 8)
            gid = base + off + lax.iota(I32, 16)
            ex = jnp.where(gid < EN, ex, 0.0)
            exv[pl.ds(off, 16)] = ex
            plsc.addupdate_scatter(denv, [dv], ex)
        return carry

    lax.fori_loop(0, NB, body_a, 0)

    def _wait(h_hbm, buf, sem):
        pltpu.make_async_copy(h_hbm.at[pl.ds(0, B)], buf, sem).wait()

    def _scale(buf, j):
        eoff = pl.multiple_of(j * B, 8)

        @plsc.parallel_loop(0, B, step=1, unroll=8)
        def scale_rows(r):
            es = plsc.load_gather(exv, [jnp.broadcast_to(eoff + r, (16,))])
            for cc in range(DH // 16):
                sl = pl.ds(cc * 16, 16)
                buf[r, sl] = buf[r, sl] * es

    for hf, h_hbm in ((0, hlo_hbm), (1, hhi_hbm)):
        pltpu.async_copy(h_hbm.at[src2v.at[0]], bufs[0], gsems[0])
        pltpu.async_copy(h_hbm.at[src2v.at[1]], bufs[1], gsems[1])

        def body_grp(g, carry):
            for b in range(3):
                i = g * 3 + b
                _wait(h_hbm, bufs[b], gsems[b])
                _scale(bufs[b], i)
                pltpu.async_copy(bufs[b], U_sh.at[dst2v.at[i]], ssems[b],
                                 add=True)
                bn = (b + 2) % 3
                j = i + 2
                if b == 0:
                    @pl.when(g > 0)
                    def _():
                        _wait(h_hbm, bufs[bn], ssems[bn])
                    pltpu.async_copy(h_hbm.at[src2v.at[j]], bufs[bn],
                                     gsems[bn])
                else:
                    @pl.when(g < NB // 3 - 1)
                    def _():
                        _wait(h_hbm, bufs[bn], ssems[bn])
                        pltpu.async_copy(h_hbm.at[src2v.at[j]], bufs[bn],
                                         gsems[bn])
            return carry

        lax.fori_loop(0, NB // 3, body_grp, 0)
        for b in range(3):
            _wait(h_hbm, bufs[b], ssems[b])
        plsc.subcore_barrier()
        pltpu.sync_copy(U_sh.at[pl.ds(sid * RPT, RPT)],
                        U_out.at[cid, pl.ds(sid * RPT, RPT), hf])
        if hf == 0:
            pltpu.sync_copy(zU_hbm, U_sh.at[pl.ds(sid * RPT, RPT)])
            plsc.subcore_barrier()
    pltpu.sync_copy(denv, den_out.at[wid])


def kernel(x, edge_index, batch, W1, a_src1, a_dst1, b1, W2, a_src2, a_dst2,
           b2, Wg, bg, Wl1, bl1, Wl2, bl2):
    zU = jnp.zeros((RPT, DH), F32)
    zden = jnp.zeros((NP,), F32)

    hlo, hhi, hs, hd, srcf, dstf = _tc_prep(x, edge_index, W1, a_src1, a_dst1)
    src2 = srcf.reshape(NW, NB, B)
    dst2 = dstf.reshape(NW, NB, B)
    U1, den1 = _sc_layer(hlo, hhi, hs, hd, src2, dst2, zU, zden)
    hlo2, hhi2, hs2, hd2 = _tc_mid(U1.reshape(NC, NP, D), den1, b1, W2,
                                   a_src2, a_dst2)
    U2, den2 = _sc_layer(hlo2, hhi2, hs2, hd2, src2, dst2, zU, zden)
    out = _tc_pool(U2.reshape(NC, NP, D), den2, b2, batch, Wg, bg, Wl1, bl1,
                   Wl2, bl2)
    return out.reshape(G)


# speedup vs baseline: 1.0141x; 1.0141x over previous
import functools

import jax
import jax.numpy as jnp
from jax import lax
from jax.experimental import pallas as pl
from jax.experimental.pallas import tpu as pltpu
from jax.experimental.pallas import tpu_sc as plsc

N = 10000
E = 320000
D = 128
DH = 64
HID = 64
G = 64

NC = 2
NS = 16
NW = NC * NS
EN = E + N
C = 10320
EN_PAD = NW * C
B = 80
NB = C // B
NP = 10240
RPT = NP // NS
F32 = jnp.float32
I32 = jnp.int32


def _leaky(v, s):
    return jnp.where(v > 0, v, s * v)


def _tc_prep(x, ei, W, a_s, a_d):
    def body(x_ref, ei_ref, w_ref, s_ref, d_ref, lo_ref, hi_ref, hs_ref,
             hd_ref, sf_ref, df_ref):
        h = jnp.dot(x_ref[...], w_ref[...], preferred_element_type=F32)
        lo_ref[...] = h[:, :DH]
        hi_ref[...] = h[:, DH:]
        hs_ref[...] = jnp.sum(h * s_ref[...], axis=1)
        hd_ref[...] = jnp.sum(h * d_ref[...], axis=1)
        loop = lax.broadcasted_iota(I32, (N,), 0)
        padz = jnp.zeros((EN_PAD - EN,), I32)
        sf_ref[...] = jnp.concatenate([ei_ref[0], loop, padz])
        df_ref[...] = jnp.concatenate([ei_ref[1], loop, padz])
    return pl.pallas_call(
        body,
        out_shape=(
            jax.ShapeDtypeStruct((N, DH), F32),
            jax.ShapeDtypeStruct((N, DH), F32),
            jax.ShapeDtypeStruct((N,), F32),
            jax.ShapeDtypeStruct((N,), F32),
            jax.ShapeDtypeStruct((EN_PAD,), I32),
            jax.ShapeDtypeStruct((EN_PAD,), I32),
        ),
    )(x, ei, W, a_s.reshape(1, D), a_d.reshape(1, D))


def _combine(u_ref, den_ref, b_ref):
    dsum = jnp.sum(den_ref[...], axis=0)[:N] + 1e-16
    usum = u_ref[0, :N] + u_ref[1, :N]
    return usum / dsum[:, None] + b_ref[...]


def _tc_mid(U, den, b, W, a_s, a_d):
    def body(u_ref, den_ref, b_ref, w_ref, s_ref, d_ref, lo_ref, hi_ref,
             hs_ref, hd_ref):
        h1 = _leaky(_combine(u_ref, den_ref, b_ref), 0.01)
        h = jnp.dot(h1, w_ref[...], preferred_element_type=F32)
        lo_ref[...] = h[:, :DH]
        hi_ref[...] = h[:, DH:]
        hs_ref[...] = jnp.sum(h * s_ref[...], axis=1)
        hd_ref[...] = jnp.sum(h * d_ref[...], axis=1)
    return pl.pallas_call(
        body,
        out_shape=(
            jax.ShapeDtypeStruct((N, DH), F32),
            jax.ShapeDtypeStruct((N, DH), F32),
            jax.ShapeDtypeStruct((N,), F32),
            jax.ShapeDtypeStruct((N,), F32),
        ),
    )(U, den, b.reshape(1, D), W, a_s.reshape(1, D), a_d.reshape(1, D))


def _tc_pool(U, den, b, batch, Wg, bg, Wl1, bl1, Wl2, bl2):
    def body(u_ref, den_ref, b_ref, bat_ref, wg_ref, bg_ref, w1_ref, b1_ref,
             w2_ref, b2_ref, o_ref):
        h2 = _leaky(_combine(u_ref, den_ref, b_ref), 0.01)
        gate = jnp.dot(h2, wg_ref[...], preferred_element_type=F32) + bg_ref[...]
        eg = jnp.exp(gate)
        gid = lax.broadcasted_iota(I32, (G, N), 0)
        M = jnp.where(gid == bat_ref[...].reshape(1, N), 1.0, 0.0)
        pden = jnp.dot(M, eg, preferred_element_type=F32,
                       precision=lax.Precision.HIGHEST) + 1e-16
        pnum = jnp.dot(M, eg * h2, preferred_element_type=F32,
                       precision=lax.Precision.HIGHEST)
        p = pnum / pden
        q = _leaky(jnp.dot(p, w1_ref[...], preferred_element_type=F32)
                   + b1_ref[...], 0.01)
        r = _leaky(jnp.dot(q, w2_ref[...], preferred_element_type=F32)
                   + b2_ref[...], 0.01)
        o_ref[...] = r
    return pl.pallas_call(
        body,
        out_shape=jax.ShapeDtypeStruct((G, 1), F32),
    )(U, den, b.reshape(1, D), batch.reshape(1, N), Wg, bg.reshape(1, D),
      Wl1, bl1.reshape(1, HID), Wl2, bl2.reshape(1, 1))


_MESH = plsc.VectorSubcoreMesh(
    core_axis_name="c", subcore_axis_name="s", num_cores=NC, num_subcores=NS)


@functools.partial(
    pl.kernel,
    out_type=(
        jax.ShapeDtypeStruct((NC, NP, 2, DH), F32),
        jax.ShapeDtypeStruct((NW, NP), F32),
    ),
    mesh=_MESH,
    compiler_params=pltpu.CompilerParams(
        needs_layout_passes=False, use_tc_tiling_on_sc=False),
    scratch_types=(
        pltpu.VMEM((N,), F32),
        pltpu.VMEM((N,), F32),
        pltpu.VMEM((NP,), F32),
        pltpu.VMEM((NB, B), I32),
        pltpu.VMEM((NB, B), I32),
        pltpu.VMEM((C,), F32),
        pltpu.VMEM((B, DH), F32),
        pltpu.VMEM((B, DH), F32),
        pltpu.VMEM((B, DH), F32),
        pltpu.VMEM_SHARED((NP, DH), F32),
        pltpu.SemaphoreType.DMA,
        pltpu.SemaphoreType.DMA,
        pltpu.SemaphoreType.DMA,
        pltpu.SemaphoreType.DMA,
        pltpu.SemaphoreType.DMA,
        pltpu.SemaphoreType.DMA,
    ),
)
def _sc_layer(hlo_hbm, hhi_hbm, asrc_hbm, adst_hbm, src2_hbm, dst2_hbm,
              zU_hbm, zden_hbm, U_out, den_out,
              asrcv, adstv, denv, src2v, dst2v, exv, rows0, rows1, rows2,
              U_sh, gs0, gs1, gs2, ss0, ss1, ss2):
    bufs = (rows0, rows1, rows2)
    gsems = (gs0, gs1, gs2)
    ssems = (ss0, ss1, ss2)
    cid = lax.axis_index("c")
    sid = lax.axis_index("s")
    wid = sid * NC + cid
    pltpu.sync_copy(asrc_hbm, asrcv)
    pltpu.sync_copy(adst_hbm, adstv)
    pltpu.sync_copy(zden_hbm, denv)
    pltpu.sync_copy(src2_hbm.at[wid], src2v)
    pltpu.sync_copy(dst2_hbm.at[wid], dst2v)
    pltpu.sync_copy(zU_hbm, U_sh.at[pl.ds(sid * RPT, RPT)])
    plsc.subcore_barrier()

    base = wid * C

    def body_a(j, carry):
        for k in range(B // 16):
            sv = src2v[j, pl.ds(k * 16, 16)]
            dv = dst2v[j, pl.ds(k * 16, 16)]
            al = plsc.load_gather(asrcv, [sv]) + plsc.load_gather(adstv, [dv])
            ex = jnp.exp(_leaky(al, 0.2))
            off = pl.multiple_of(j * B + k * 16, 8)
            gid = base + off + lax.iota(I32, 16)
            ex = jnp.where(gid < EN, ex, 0.0)
            exv[pl.ds(off, 16)] = ex
            plsc.addupdate_scatter(denv, [dv], ex)
        return carry

    lax.fori_loop(0, NB, body_a, 0)

    def _wait(h_hbm, buf, sem):
        pltpu.make_async_copy(h_hbm.at[pl.ds(0, B)], buf, sem).wait()

    def _scale(buf, j):
        eoff = pl.multiple_of(j * B, 8)

        @plsc.parallel_loop(0, B, step=1, unroll=8)
        def scale_rows(r):
            es = plsc.load_gather(exv, [jnp.broadcast_to(eoff + r, (16,))])
            for cc in range(DH // 16):
                sl = pl.ds(cc * 16, 16)
                buf[r, sl] = buf[r, sl] * es

    for hf, h_hbm in ((0, hlo_hbm), (1, hhi_hbm)):
        pltpu.async_copy(h_hbm.at[src2v.at[0]], bufs[0], gsems[0])
        pltpu.async_copy(h_hbm.at[src2v.at[1]], bufs[1], gsems[1])

        def body_grp(g, carry):
            for b in range(3):
                i = g * 3 + b
                _wait(h_hbm, bufs[b], gsems[b])
                _scale(bufs[b], i)
                pltpu.async_copy(bufs[b], U_sh.at[dst2v.at[i]], ssems[b],
                                 add=True)
                bn = (b + 2) % 3
                j = i + 2
                if b == 0:
                    @pl.when(g > 0)
                    def _():
                        _wait(h_hbm, bufs[bn], ssems[bn])
                    pltpu.async_copy(h_hbm.at[src2v.at[j]], bufs[bn],
                                     gsems[bn])
                else:
                    @pl.when(g < NB // 3 - 1)
                    def _():
                        _wait(h_hbm, bufs[bn], ssems[bn])
                        pltpu.async_copy(h_hbm.at[src2v.at[j]], bufs[bn],
                                         gsems[bn])
            return carry

        lax.fori_loop(0, NB // 3, body_grp, 0)
        for b in range(3):
            _wait(h_hbm, bufs[b], ssems[b])
        plsc.subcore_barrier()
        pltpu.sync_copy(U_sh.at[pl.ds(sid * RPT, RPT)],
                        U_out.at[cid, pl.ds(sid * RPT, RPT), hf])
        if hf == 0:
            pltpu.sync_copy(zU_hbm, U_sh.at[pl.ds(sid * RPT, RPT)])
            plsc.subcore_barrier()
    pltpu.sync_copy(denv, den_out.at[wid])


def kernel(x, edge_index, batch, W1, a_src1, a_dst1, b1, W2, a_src2, a_dst2,
           b2, Wg, bg, Wl1, bl1, Wl2, bl2):
    zU = jnp.zeros((RPT, DH), F32)
    zden = jnp.zeros((NP,), F32)

    hlo, hhi, hs, hd, srcf, dstf = _tc_prep(x, edge_index, W1, a_src1, a_dst1)
    src2 = srcf.reshape(NW, NB, B)
    dst2 = dstf.reshape(NW, NB, B)
    U1, den1 = _sc_layer(hlo, hhi, hs, hd, src2, dst2, zU, zden)
    hlo2, hhi2, hs2, hd2 = _tc_mid(U1.reshape(NC, NP, D), den1, b1, W2,
                                   a_src2, a_dst2)
    U2, den2 = _sc_layer(hlo2, hhi2, hs2, hd2, src2, dst2, zU, zden)
    out = _tc_pool(U2.reshape(NC, NP, D), den2, b2, batch, Wg, bg, Wl1, bl1,
                   Wl2, bl2)
    return out.reshape(G)


# speedup vs baseline: 1.3880x; 1.3687x over previous
import functools

import jax
import jax.numpy as jnp
from jax import lax
from jax.experimental import pallas as pl
from jax.experimental.pallas import tpu as pltpu
from jax.experimental.pallas import tpu_sc as plsc

N = 10000
E = 320000
D = 128
DH = 64
HID = 64
G = 64

NC = 2
NS = 16
NW = NC * NS
EN = E + N
C = 10320
EN_PAD = NW * C
B = 80
NB = C // B
NP = 10240
RPT = NP // NS
F32 = jnp.float32
I32 = jnp.int32


def _leaky(v, s):
    return jnp.where(v > 0, v, s * v)


def _tc_prep(x, ei, W, a_s, a_d):
    def body(x_ref, ei_ref, w_ref, s_ref, d_ref, h_ref, hs_ref,
             hd_ref, sf_ref, df_ref):
        h = jnp.dot(x_ref[...], w_ref[...], preferred_element_type=F32)
        h_ref[...] = h
        hs_ref[...] = jnp.sum(h * s_ref[...], axis=1)
        hd_ref[...] = jnp.sum(h * d_ref[...], axis=1)
        loop = lax.broadcasted_iota(I32, (N,), 0)
        padz = jnp.zeros((EN_PAD - EN,), I32)
        sf_ref[...] = jnp.concatenate([ei_ref[0], loop, padz])
        df_ref[...] = jnp.concatenate([ei_ref[1], loop, padz])
    return pl.pallas_call(
        body,
        out_shape=(
            jax.ShapeDtypeStruct((N, D), F32),
            jax.ShapeDtypeStruct((N,), F32),
            jax.ShapeDtypeStruct((N,), F32),
            jax.ShapeDtypeStruct((EN_PAD,), I32),
            jax.ShapeDtypeStruct((EN_PAD,), I32),
        ),
    )(x, ei, W, a_s.reshape(1, D), a_d.reshape(1, D))


def _combine(u_ref, den_ref, b_ref):
    dsum = jnp.sum(den_ref[...], axis=0)[:N] + 1e-16
    usum = u_ref[0, :N] + u_ref[1, :N]
    return usum / dsum[:, None] + b_ref[...]


def _tc_mid(U, den, b, W, a_s, a_d):
    def body(u_ref, den_ref, b_ref, w_ref, s_ref, d_ref, h_ref,
             hs_ref, hd_ref):
        h1 = _leaky(_combine(u_ref, den_ref, b_ref), 0.01)
        h = jnp.dot(h1, w_ref[...], preferred_element_type=F32)
        h_ref[...] = h
        hs_ref[...] = jnp.sum(h * s_ref[...], axis=1)
        hd_ref[...] = jnp.sum(h * d_ref[...], axis=1)
    return pl.pallas_call(
        body,
        out_shape=(
            jax.ShapeDtypeStruct((N, D), F32),
            jax.ShapeDtypeStruct((N,), F32),
            jax.ShapeDtypeStruct((N,), F32),
        ),
    )(U, den, b.reshape(1, D), W, a_s.reshape(1, D), a_d.reshape(1, D))


def _tc_pool(U, den, b, batch, Wg, bg, Wl1, bl1, Wl2, bl2):
    def body(u_ref, den_ref, b_ref, bat_ref, wg_ref, bg_ref, w1_ref, b1_ref,
             w2_ref, b2_ref, o_ref):
        h2 = _leaky(_combine(u_ref, den_ref, b_ref), 0.01)
        gate = jnp.dot(h2, wg_ref[...], preferred_element_type=F32) + bg_ref[...]
        eg = jnp.exp(gate)
        gid = lax.broadcasted_iota(I32, (G, N), 0)
        M = jnp.where(gid == bat_ref[...].reshape(1, N), 1.0, 0.0)
        pden = jnp.dot(M, eg, preferred_element_type=F32,
                       precision=lax.Precision.HIGHEST) + 1e-16
        pnum = jnp.dot(M, eg * h2, preferred_element_type=F32,
                       precision=lax.Precision.HIGHEST)
        p = pnum / pden
        q = _leaky(jnp.dot(p, w1_ref[...], preferred_element_type=F32)
                   + b1_ref[...], 0.01)
        r = _leaky(jnp.dot(q, w2_ref[...], preferred_element_type=F32)
                   + b2_ref[...], 0.01)
        o_ref[...] = r
    return pl.pallas_call(
        body,
        out_shape=jax.ShapeDtypeStruct((G, 1), F32),
    )(U, den, b.reshape(1, D), batch.reshape(1, N), Wg, bg.reshape(1, D),
      Wl1, bl1.reshape(1, HID), Wl2, bl2.reshape(1, 1))


_MESH = plsc.VectorSubcoreMesh(
    core_axis_name="c", subcore_axis_name="s", num_cores=NC, num_subcores=NS)


@functools.partial(
    pl.kernel,
    out_type=(
        jax.ShapeDtypeStruct((NC, NP, D), F32),
        jax.ShapeDtypeStruct((NW, NP), F32),
    ),
    mesh=_MESH,
    compiler_params=pltpu.CompilerParams(
        needs_layout_passes=False, use_tc_tiling_on_sc=False),
    scratch_types=(
        pltpu.VMEM((N,), F32),
        pltpu.VMEM((N,), F32),
        pltpu.VMEM((NP,), F32),
        pltpu.VMEM((C,), I32),
        pltpu.VMEM((C,), I32),
        pltpu.VMEM((NB, B), I32),
        pltpu.VMEM((C,), F32),
        pltpu.VMEM((B, DH), F32),
        pltpu.VMEM((B, DH), F32),
        pltpu.VMEM((B, DH), F32),
        pltpu.VMEM_SHARED((NP, DH), F32),
        pltpu.SemaphoreType.DMA,
        pltpu.SemaphoreType.DMA,
        pltpu.SemaphoreType.DMA,
        pltpu.SemaphoreType.DMA,
        pltpu.SemaphoreType.DMA,
        pltpu.SemaphoreType.DMA,
    ),
)
def _sc_layer(h2_hbm, asrc_hbm, adst_hbm, srcf_hbm, dstf_hbm,
              zU_hbm, zden_hbm, U_out, den_out,
              asrcv, adstv, denv, srclo, srchi, dst2v, exv,
              rows0, rows1, rows2, U_sh, gs0, gs1, gs2, ss0, ss1, ss2):
    bufs = (rows0, rows1, rows2)
    gsems = (gs0, gs1, gs2)
    ssems = (ss0, ss1, ss2)
    cid = lax.axis_index("c")
    sid = lax.axis_index("s")
    wid = sid * NC + cid
    base = wid * C

    def fill_dst(j, carry):
        pltpu.async_copy(dstf_hbm.at[pl.ds(base + j * B, B)], dst2v.at[j],
                         gs0)
        return carry

    lax.fori_loop(0, NB, fill_dst, 0)
    pltpu.sync_copy(asrc_hbm, asrcv)
    pltpu.sync_copy(adst_hbm, adstv)
    pltpu.sync_copy(zden_hbm, denv)
    pltpu.sync_copy(srcf_hbm.at[pl.ds(base, C)], srclo)
    pltpu.sync_copy(zU_hbm, U_sh.at[pl.ds(sid * RPT, RPT)])

    def drain_dst(j, carry):
        pltpu.make_async_copy(dstf_hbm.at[pl.ds(0, B)], dst2v.at[0],
                              gs0).wait()
        return carry

    lax.fori_loop(0, NB, drain_dst, 0)
    plsc.subcore_barrier()

    def body_a(j, carry):
        for k in range(B // 16):
            off = pl.multiple_of(j * B + k * 16, 8)
            sv = srclo[pl.ds(off, 16)]
            dv = dst2v[j, pl.ds(k * 16, 16)]
            s2 = sv + sv
            srclo[pl.ds(off, 16)] = s2
            srchi[pl.ds(off, 16)] = s2 + 1
            al = plsc.load_gather(asrcv, [sv]) + plsc.load_gather(adstv, [dv])
            ex = jnp.exp(_leaky(al, 0.2))
            gid = base + off + lax.iota(I32, 16)
            ex = jnp.where(gid < EN, ex, 0.0)
            exv[pl.ds(off, 16)] = ex
            plsc.addupdate_scatter(denv, [dv], ex)
        return carry

    lax.fori_loop(0, NB, body_a, 0)

    def _wait(buf, sem):
        pltpu.make_async_copy(h2_hbm.at[pl.ds(0, B)], buf, sem).wait()

    def _scale(buf, j):
        eoff = pl.multiple_of(j * B, 8)

        @plsc.parallel_loop(0, B, step=1, unroll=8)
        def scale_rows(r):
            es = plsc.load_gather(exv, [jnp.broadcast_to(eoff + r, (16,))])
            for cc in range(DH // 16):
                sl = pl.ds(cc * 16, 16)
                buf[r, sl] = buf[r, sl] * es

    for hf, sidx in ((0, srclo), (1, srchi)):
        pltpu.async_copy(h2_hbm.at[sidx.at[pl.ds(0, B)]], bufs[0], gsems[0])
        pltpu.async_copy(h2_hbm.at[sidx.at[pl.ds(B, B)]], bufs[1], gsems[1])

        def body_grp(g, carry):
            for b in range(3):
                i = g * 3 + b
                _wait(bufs[b], gsems[b])
                _scale(bufs[b], i)
                pltpu.async_copy(bufs[b], U_sh.at[dst2v.at[i]], ssems[b],
                                 add=True)
                bn = (b + 2) % 3
                j = i + 2
                if b == 0:
                    @pl.when(g > 0)
                    def _():
                        _wait(bufs[bn], ssems[bn])
                    pltpu.async_copy(
                        h2_hbm.at[sidx.at[pl.ds(pl.multiple_of(j * B, 8), B)]],
                        bufs[bn], gsems[bn])
                else:
                    @pl.when(g < NB // 3 - 1)
                    def _():
                        _wait(bufs[bn], ssems[bn])
                        pltpu.async_copy(
                            h2_hbm.at[
                                sidx.at[pl.ds(pl.multiple_of(j * B, 8), B)]],
                            bufs[bn], gsems[bn])
            return carry

        lax.fori_loop(0, NB // 3, body_grp, 0)
        for b in range(3):
            _wait(bufs[b], ssems[b])
        plsc.subcore_barrier()
        pltpu.sync_copy(U_sh.at[pl.ds(sid * RPT, RPT)],
                        U_out.at[cid, pl.ds(sid * RPT, RPT),
                                 pl.ds(hf * DH, DH)])
        if hf == 0:
            pltpu.sync_copy(zU_hbm, U_sh.at[pl.ds(sid * RPT, RPT)])
            plsc.subcore_barrier()
    pltpu.sync_copy(denv, den_out.at[wid])


def kernel(x, edge_index, batch, W1, a_src1, a_dst1, b1, W2, a_src2, a_dst2,
           b2, Wg, bg, Wl1, bl1, Wl2, bl2):
    zU = jnp.zeros((RPT, DH), F32)
    zden = jnp.zeros((NP,), F32)

    h1, hs, hd, srcf, dstf = _tc_prep(x, edge_index, W1, a_src1, a_dst1)
    U1, den1 = _sc_layer(h1.reshape(2 * N, DH), hs, hd, srcf, dstf, zU, zden)
    h2, hs2, hd2 = _tc_mid(U1, den1, b1, W2, a_src2, a_dst2)
    U2, den2 = _sc_layer(h2.reshape(2 * N, DH), hs2, hd2, srcf, dstf, zU,
                         zden)
    out = _tc_pool(U2, den2, b2, batch, Wg, bg, Wl1, bl1, Wl2, bl2)
    return out.reshape(G)
